# baseline (speedup 1.0000x reference)
import jax
import jax.numpy as jnp
from jax.experimental import pallas as pl
from jax.experimental.pallas import tpu as pltpu

HIDDEN = 256
EPS = 1e-6
RSPLIT = 2


def _block_kernel(x_ref, p_ref, w_ref, b_ref, g1_ref, b1_ref, g2_ref, b2_ref,
                  o_ref):
    x = x_ref[0]
    p = p_ref[0]
    inv_h = 1.0 / x.shape[-1]
    half = p.shape[0]
    r_id = pl.program_id(1)
    row0 = r_id * half

    s1 = jnp.sum(x, axis=-1, keepdims=True)
    s2 = jnp.sum(x * x, axis=-1, keepdims=True)
    mu = s1 * inv_h
    r = jax.lax.rsqrt(s2 * inv_h - mu * mu + EPS)
    h = ((x - mu) * r) * g1_ref[0] + b1_ref[0]

    agg = jnp.dot(p, h, preferred_element_type=jnp.float32)
    conv = jnp.maximum(
        jnp.dot(agg, w_ref[...], preferred_element_type=jnp.float32)
        + b_ref[0], 0.0)

    y = x_ref[0, pl.ds(row0, half), :] + conv
    mu2 = jnp.sum(y, axis=-1, keepdims=True) * inv_h
    n2 = jnp.sum(y * y, axis=-1, keepdims=True) * inv_h
    r2 = jax.lax.rsqrt(n2 - mu2 * mu2 + EPS)
    o_ref[0] = ((y - mu2) * r2) * g2_ref[0] + b2_ref[0]


def kernel(x, mask, inputP, W, b, ln1_g, ln1_b, ln2_g, ln2_b):
    del mask
    B, N, H = x.shape
    HN = N // RSPLIT

    vec = lambda v: v.reshape(1, H)
    return pl.pallas_call(
        _block_kernel,
        grid=(B, RSPLIT),
        in_specs=[
            pl.BlockSpec((1, N, H), lambda i, r: (i, 0, 0)),
            pl.BlockSpec((1, HN, N), lambda i, r: (i, r, 0)),
            pl.BlockSpec((H, H), lambda i, r: (0, 0)),
            pl.BlockSpec((1, H), lambda i, r: (0, 0)),
            pl.BlockSpec((1, H), lambda i, r: (0, 0)),
            pl.BlockSpec((1, H), lambda i, r: (0, 0)),
            pl.BlockSpec((1, H), lambda i, r: (0, 0)),
            pl.BlockSpec((1, H), lambda i, r: (0, 0)),
        ],
        out_specs=pl.BlockSpec((1, HN, H), lambda i, r: (i, r, 0)),
        out_shape=jax.ShapeDtypeStruct((B, N, H), x.dtype),
        compiler_params=pltpu.CompilerParams(
            dimension_semantics=("arbitrary", "arbitrary")),
    )(x, inputP, W, vec(b), vec(ln1_g), vec(ln1_b), vec(ln2_g), vec(ln2_b))


# speedup vs baseline: 1.2560x; 1.2560x over previous
import jax
import jax.numpy as jnp
from jax.experimental import pallas as pl
from jax.experimental.pallas import tpu as pltpu

HIDDEN = 256
EPS = 1e-6
PDEPTH = 3
XDEPTH = 2


def _compute_batch(x, p, w_ref, b_ref, g1_ref, b1_ref, g2_ref, b2_ref):
    inv_h = 1.0 / x.shape[-1]
    s1 = jnp.sum(x, axis=-1, keepdims=True)
    s2 = jnp.sum(x * x, axis=-1, keepdims=True)
    mu = s1 * inv_h
    r = jax.lax.rsqrt(s2 * inv_h - mu * mu + EPS)
    h = ((x - mu) * r) * g1_ref[0] + b1_ref[0]

    agg = jnp.dot(p, h, preferred_element_type=jnp.float32)
    conv = jnp.maximum(
        jnp.dot(agg, w_ref[...], preferred_element_type=jnp.float32)
        + b_ref[0], 0.0)

    y = x + conv
    mu2 = jnp.sum(y, axis=-1, keepdims=True) * inv_h
    n2 = jnp.sum(y * y, axis=-1, keepdims=True) * inv_h
    r2 = jax.lax.rsqrt(n2 - mu2 * mu2 + EPS)
    return ((y - mu2) * r2) * g2_ref[0] + b2_ref[0]


def _pipeline_kernel(x_hbm, p_hbm, w_ref, b_ref, g1_ref, b1_ref, g2_ref,
                     b2_ref, o_hbm, x_buf, p_buf, o_buf,
                     x_sem, p_sem, o_sem):
    B = x_hbm.shape[0]

    def x_copy(b):
        return pltpu.make_async_copy(
            x_hbm.at[b], x_buf.at[b % XDEPTH], x_sem.at[b % XDEPTH])

    def p_copy(b):
        return pltpu.make_async_copy(
            p_hbm.at[b], p_buf.at[b % PDEPTH], p_sem.at[b % PDEPTH])

    def o_copy(b):
        return pltpu.make_async_copy(
            o_buf.at[b % XDEPTH], o_hbm.at[b], o_sem.at[b % XDEPTH])

    x_copy(0).start()
    p_copy(0).start()
    x_copy(1).start()
    p_copy(1).start()
    p_copy(2).start()

    for b in range(B):
        x_copy(b).wait()
        p_copy(b).wait()
        if b >= XDEPTH:
            o_copy(b - XDEPTH).wait()
        out = _compute_batch(x_buf[b % XDEPTH], p_buf[b % PDEPTH],
                             w_ref, b_ref, g1_ref, b1_ref, g2_ref, b2_ref)
        o_buf[b % XDEPTH] = out
        o_copy(b).start()
        if b + XDEPTH < B:
            x_copy(b + XDEPTH).start()
        if b + PDEPTH < B:
            p_copy(b + PDEPTH).start()

    o_copy(B - 2).wait()
    o_copy(B - 1).wait()


def kernel(x, mask, inputP, W, b, ln1_g, ln1_b, ln2_g, ln2_b):
    del mask
    B, N, H = x.shape

    vec = lambda v: v.reshape(1, H)
    return pl.pallas_call(
        _pipeline_kernel,
        in_specs=[
            pl.BlockSpec(memory_space=pltpu.HBM),
            pl.BlockSpec(memory_space=pltpu.HBM),
            pl.BlockSpec((H, H), lambda: (0, 0)),
            pl.BlockSpec((1, H), lambda: (0, 0)),
            pl.BlockSpec((1, H), lambda: (0, 0)),
            pl.BlockSpec((1, H), lambda: (0, 0)),
            pl.BlockSpec((1, H), lambda: (0, 0)),
            pl.BlockSpec((1, H), lambda: (0, 0)),
        ],
        out_specs=pl.BlockSpec(memory_space=pltpu.HBM),
        out_shape=jax.ShapeDtypeStruct((B, N, H), x.dtype),
        scratch_shapes=[
            pltpu.VMEM((XDEPTH, N, H), jnp.float32),
            pltpu.VMEM((PDEPTH, N, N), jnp.float32),
            pltpu.VMEM((XDEPTH, N, H), jnp.float32),
            pltpu.SemaphoreType.DMA((XDEPTH,)),
            pltpu.SemaphoreType.DMA((PDEPTH,)),
            pltpu.SemaphoreType.DMA((XDEPTH,)),
        ],
    )(x, inputP, W, vec(b), vec(ln1_g), vec(ln1_b), vec(ln2_g), vec(ln2_b))


# speedup vs baseline: 1.3092x; 1.0424x over previous
import jax
import jax.numpy as jnp
from jax.experimental import pallas as pl
from jax.experimental.pallas import tpu as pltpu

HIDDEN = 256
EPS = 1e-6


def _block_kernel(x_ref, p_ref, w_ref, b_ref, g1_ref, b1_ref, g2_ref, b2_ref,
                  o_ref):
    x = x_ref[0]
    p = p_ref[0]
    inv_h = 1.0 / x.shape[-1]

    s1 = jnp.sum(x, axis=-1, keepdims=True)
    s2 = jnp.sum(x * x, axis=-1, keepdims=True)
    mu = s1 * inv_h
    r = jax.lax.rsqrt(s2 * inv_h - mu * mu + EPS)
    h = ((x - mu) * r) * g1_ref[0] + b1_ref[0]

    h2 = jnp.dot(h, w_ref[...], preferred_element_type=jnp.float32)
    conv = jnp.maximum(
        jnp.dot(p, h2, preferred_element_type=jnp.float32) + b_ref[0], 0.0)

    y = x + conv
    mu2 = jnp.sum(y, axis=-1, keepdims=True) * inv_h
    n2 = jnp.sum(y * y, axis=-1, keepdims=True) * inv_h
    r2 = jax.lax.rsqrt(n2 - mu2 * mu2 + EPS)
    o_ref[0] = ((y - mu2) * r2) * g2_ref[0] + b2_ref[0]


def kernel(x, mask, inputP, W, b, ln1_g, ln1_b, ln2_g, ln2_b):
    del mask
    B, N, H = x.shape

    vec = lambda v: v.reshape(1, H)
    return pl.pallas_call(
        _block_kernel,
        grid=(B,),
        in_specs=[
            pl.BlockSpec((1, N, H), lambda i: (i, 0, 0)),
            pl.BlockSpec((1, N, N), lambda i: (i, 0, 0)),
            pl.BlockSpec((H, H), lambda i: (0, 0)),
            pl.BlockSpec((1, H), lambda i: (0, 0)),
            pl.BlockSpec((1, H), lambda i: (0, 0)),
            pl.BlockSpec((1, H), lambda i: (0, 0)),
            pl.BlockSpec((1, H), lambda i: (0, 0)),
            pl.BlockSpec((1, H), lambda i: (0, 0)),
        ],
        out_specs=pl.BlockSpec((1, N, H), lambda i: (i, 0, 0)),
        out_shape=jax.ShapeDtypeStruct((B, N, H), x.dtype),
        compiler_params=pltpu.CompilerParams(
            dimension_semantics=("arbitrary",)),
    )(x, inputP, W, vec(b), vec(ln1_g), vec(ln1_b), vec(ln2_g), vec(ln2_b))
